# baseline (speedup 1.0000x reference)
import functools

import jax
import jax.numpy as jnp
from jax import lax
from jax.experimental import pallas as pl
from jax.experimental.pallas import tpu as pltpu
from jax.experimental.pallas import tpu_sc as plsc

_L = 16
_NW = 32
_CH = 8192


def _sc_body(img_h, mask_h, boost_h, ab_h, out_h, img_v, mask_v, boost_v,
             out_v, ab_v, plane_elems, half_elems):
    cid = lax.axis_index("c")
    sid = lax.axis_index("s")
    w = sid * 2 + cid

    pltpu.sync_copy(ab_h, ab_v)
    a = ab_v[pl.ds(0, _L)]
    b = ab_v[pl.ds(_L, _L)]

    n_chunks = half_elems // _CH
    n_vec = _CH // _L
    unroll = 16

    for t in range(3):
        h = w + t * _NW
        p = h // 2
        img_off = p * plane_elems + (h % 2) * half_elems
        mask_off = (p // 3) * plane_elems + (h % 2) * half_elems

        def chunk_body(i, _, img_off=img_off, mask_off=mask_off):
            io = img_off + i * _CH
            mo = mask_off + i * _CH
            pltpu.sync_copy(img_h.at[pl.ds(io, _CH)], img_v)
            pltpu.sync_copy(mask_h.at[pl.ds(mo, _CH)], mask_v)
            pltpu.sync_copy(boost_h.at[pl.ds(io, _CH)], boost_v)

            def vec_body(j, _):
                base = j * (unroll * _L)
                for u in range(unroll):
                    s = pl.ds(base + u * _L, _L)
                    bt = boost_v[s] * a + b
                    out_v[s] = jnp.where(mask_v[s] < 0.5, bt, img_v[s])
                return 0

            lax.fori_loop(0, n_vec // unroll, vec_body, 0)
            pltpu.sync_copy(out_v, out_h.at[pl.ds(io, _CH)])
            return 0

        lax.fori_loop(0, n_chunks, chunk_body, 0)


def kernel(standardization, batchimg, batchmask, boost):
    batchimg = batchimg.astype(jnp.float32)
    batchmask = batchmask.astype(jnp.float32)
    boost = boost.astype(jnp.float32)
    B, C, H, W = batchimg.shape
    std = jnp.asarray(standardization)
    a = jnp.where(std != 0, jnp.float32(1.0 / 3.9), jnp.float32(0.5 / 3.9))
    b = jnp.where(std != 0, jnp.float32(0.0), jnp.float32(0.5))
    ab = jnp.concatenate([jnp.full((_L,), a, jnp.float32),
                          jnp.full((_L,), b, jnp.float32)])

    n = B * C * H * W
    plane_elems = H * W
    half_elems = plane_elems // 2

    f = functools.partial(
        pl.kernel,
        out_type=jax.ShapeDtypeStruct((n,), jnp.float32),
        mesh=plsc.VectorSubcoreMesh(core_axis_name="c", subcore_axis_name="s"),
        scratch_types=[
            pltpu.VMEM((_CH,), jnp.float32),
            pltpu.VMEM((_CH,), jnp.float32),
            pltpu.VMEM((_CH,), jnp.float32),
            pltpu.VMEM((_CH,), jnp.float32),
            pltpu.VMEM((2 * _L,), jnp.float32),
        ],
    )(functools.partial(_sc_body, plane_elems=plane_elems,
                        half_elems=half_elems))
    out = f(batchimg.reshape(-1), batchmask.reshape(-1), boost.reshape(-1), ab)
    return out.reshape(B, C, H, W)


# speedup vs baseline: 1.3460x; 1.3460x over previous
import functools

import jax
import jax.numpy as jnp
from jax import lax
from jax.experimental import pallas as pl
from jax.experimental.pallas import tpu as pltpu
from jax.experimental.pallas import tpu_sc as plsc

_L = 16
_NW = 32
_CH = 8192
_UNROLL = 16


def _sc_body(img_h, mask_h, boost_h, ab_h, out_h, img_v, mask_v, boost_v,
             out_v, ab_v, si0, si1, so0, so1, plane_elems, half_elems):
    cid = lax.axis_index("c")
    sid = lax.axis_index("s")
    w = sid * 2 + cid

    pltpu.sync_copy(ab_h, ab_v)
    a = ab_v[pl.ds(0, _L)]
    b = ab_v[pl.ds(_L, _L)]

    n_chunks = half_elems // _CH
    sem_in = (si0, si1)
    sem_out = (so0, so1)

    def in_copies(io, mo, b):
        return (
            pltpu.make_async_copy(img_h.at[pl.ds(io, _CH)], img_v.at[b],
                                  sem_in[b]),
            pltpu.make_async_copy(mask_h.at[pl.ds(mo, _CH)], mask_v.at[b],
                                  sem_in[b]),
            pltpu.make_async_copy(boost_h.at[pl.ds(io, _CH)], boost_v.at[b],
                                  sem_in[b]),
        )

    for t in range(3):
        h = w + t * _NW
        p = h // 2
        img_off = p * plane_elems + (h % 2) * half_elems
        mask_off = (p // 3) * plane_elems + (h % 2) * half_elems

        for c0 in in_copies(img_off, mask_off, 0):
            c0.start()

        def k_body(k, _, img_off=img_off, mask_off=mask_off):
            for bb_ in (0, 1):
                c = 2 * k + bb_
                io = img_off + c * _CH
                mo = mask_off + c * _CH
                for cp in in_copies(io, mo, bb_):
                    cp.wait()

                @pl.when(c + 1 < n_chunks)
                def _():
                    nio = img_off + (c + 1) * _CH
                    nmo = mask_off + (c + 1) * _CH
                    for cp in in_copies(nio, nmo, 1 - bb_):
                        cp.start()

                @pl.when(c >= 2)
                def _():
                    pltpu.make_async_copy(
                        out_v.at[bb_], out_h.at[pl.ds(io, _CH)],
                        sem_out[bb_]).wait()

                ib, mb, bs, ob = (img_v.at[bb_], mask_v.at[bb_],
                                  boost_v.at[bb_], out_v.at[bb_])

                def vec_body(j, _):
                    base = j * (_UNROLL * _L)
                    for u in range(_UNROLL):
                        s = pl.ds(base + u * _L, _L)
                        bt = bs[s] * a + b
                        ob[s] = jnp.where(mb[s] < 0.5, bt, ib[s])
                    return 0

                lax.fori_loop(0, _CH // (_UNROLL * _L), vec_body, 0)

                pltpu.make_async_copy(out_v.at[bb_],
                                      out_h.at[pl.ds(io, _CH)],
                                      sem_out[bb_]).start()
            return 0

        lax.fori_loop(0, n_chunks // 2, k_body, 0)

        for bb_ in (0, 1):
            c = n_chunks - 2 + bb_
            io = img_off + c * _CH
            pltpu.make_async_copy(out_v.at[bb_], out_h.at[pl.ds(io, _CH)],
                                  sem_out[bb_]).wait()


def kernel(standardization, batchimg, batchmask, boost):
    batchimg = batchimg.astype(jnp.float32)
    batchmask = batchmask.astype(jnp.float32)
    boost = boost.astype(jnp.float32)
    B, C, H, W = batchimg.shape
    std = jnp.asarray(standardization)
    a = jnp.where(std != 0, jnp.float32(1.0 / 3.9), jnp.float32(0.5 / 3.9))
    b = jnp.where(std != 0, jnp.float32(0.0), jnp.float32(0.5))
    ab = jnp.concatenate([jnp.full((_L,), a, jnp.float32),
                          jnp.full((_L,), b, jnp.float32)])

    n = B * C * H * W
    plane_elems = H * W
    half_elems = plane_elems // 2

    f = functools.partial(
        pl.kernel,
        out_type=jax.ShapeDtypeStruct((n,), jnp.float32),
        mesh=plsc.VectorSubcoreMesh(core_axis_name="c", subcore_axis_name="s"),
        scratch_types=[
            pltpu.VMEM((2, _CH), jnp.float32),
            pltpu.VMEM((2, _CH), jnp.float32),
            pltpu.VMEM((2, _CH), jnp.float32),
            pltpu.VMEM((2, _CH), jnp.float32),
            pltpu.VMEM((2 * _L,), jnp.float32),
            pltpu.SemaphoreType.DMA,
            pltpu.SemaphoreType.DMA,
            pltpu.SemaphoreType.DMA,
            pltpu.SemaphoreType.DMA,
        ],
    )(functools.partial(_sc_body, plane_elems=plane_elems,
                        half_elems=half_elems))
    out = f(batchimg.reshape(-1), batchmask.reshape(-1), boost.reshape(-1), ab)
    return out.reshape(B, C, H, W)


# speedup vs baseline: 3.1395x; 2.3324x over previous
import functools

import jax
import jax.numpy as jnp
from jax import lax
from jax.experimental import pallas as pl
from jax.experimental.pallas import tpu as pltpu
from jax.experimental.pallas import tpu_sc as plsc

_L = 16
_NW = 32
_RR = 16


def _sc_body(img_h, mask_h, boost_h, ab_h, out_h, img_v, mask_v, boost_v,
             out_v, ab_v, si0, si1, so0, so1, rows_half, W):
    cid = lax.axis_index("c")
    sid = lax.axis_index("s")
    w = sid * 2 + cid

    pltpu.sync_copy(ab_h, ab_v)
    a = ab_v[pl.ds(0, _L)]
    b = ab_v[pl.ds(_L, _L)]

    n_chunks = rows_half // _RR
    sem_in = (si0, si1)
    sem_out = (so0, so1)
    n_vec = W // _L

    def in_copies(ir, mr, b_):
        return (
            pltpu.make_async_copy(img_h.at[pl.ds(ir, _RR)], img_v.at[b_],
                                  sem_in[b_]),
            pltpu.make_async_copy(mask_h.at[pl.ds(mr, _RR)], mask_v.at[b_],
                                  sem_in[b_]),
            pltpu.make_async_copy(boost_h.at[pl.ds(ir, _RR)], boost_v.at[b_],
                                  sem_in[b_]),
        )

    for t in range(3):
        h = w + t * _NW
        p = h // 2
        img_base = p * (2 * rows_half) + (h % 2) * rows_half
        mask_base = (p // 3) * (2 * rows_half) + (h % 2) * rows_half

        for cp in in_copies(img_base, mask_base, 0):
            cp.start()

        def k_body(k, _, img_base=img_base, mask_base=mask_base):
            for b_ in (0, 1):
                c = 2 * k + b_
                ir = img_base + c * _RR
                mr = mask_base + c * _RR
                for cp in in_copies(ir, mr, b_):
                    cp.wait()

                @pl.when(c + 1 < n_chunks)
                def _():
                    for cp in in_copies(ir + _RR, mr + _RR, 1 - b_):
                        cp.start()

                @pl.when(c >= 2)
                def _():
                    pltpu.make_async_copy(out_v.at[b_],
                                          out_h.at[pl.ds(ir, _RR)],
                                          sem_out[b_]).wait()

                ib, mb, bs, ob = (img_v.at[b_], mask_v.at[b_],
                                  boost_v.at[b_], out_v.at[b_])

                def row_body(r, _):
                    for cc in range(n_vec):
                        s = pl.ds(cc * _L, _L)
                        bt = bs[r, s] * a + b
                        ob[r, s] = jnp.where(mb[r, s] < 0.5, bt, ib[r, s])
                    return 0

                lax.fori_loop(0, _RR, row_body, 0)

                pltpu.make_async_copy(out_v.at[b_],
                                      out_h.at[pl.ds(ir, _RR)],
                                      sem_out[b_]).start()
            return 0

        lax.fori_loop(0, n_chunks // 2, k_body, 0)

        for b_ in (0, 1):
            c = n_chunks - 2 + b_
            ir = img_base + c * _RR
            pltpu.make_async_copy(out_v.at[b_], out_h.at[pl.ds(ir, _RR)],
                                  sem_out[b_]).wait()


def kernel(standardization, batchimg, batchmask, boost):
    batchimg = batchimg.astype(jnp.float32)
    batchmask = batchmask.astype(jnp.float32)
    boost = boost.astype(jnp.float32)
    B, C, H, W = batchimg.shape
    std = jnp.asarray(standardization)
    a = jnp.where(std != 0, jnp.float32(1.0 / 3.9), jnp.float32(0.5 / 3.9))
    b = jnp.where(std != 0, jnp.float32(0.0), jnp.float32(0.5))
    ab = jnp.concatenate([jnp.full((_L,), a, jnp.float32),
                          jnp.full((_L,), b, jnp.float32)])

    rows_half = H // 2

    f = functools.partial(
        pl.kernel,
        out_type=jax.ShapeDtypeStruct((B * C * H, W), jnp.float32),
        mesh=plsc.VectorSubcoreMesh(core_axis_name="c", subcore_axis_name="s"),
        compiler_params=pltpu.CompilerParams(use_tc_tiling_on_sc=True),
        scratch_types=[
            pltpu.VMEM((2, _RR, W), jnp.float32),
            pltpu.VMEM((2, _RR, W), jnp.float32),
            pltpu.VMEM((2, _RR, W), jnp.float32),
            pltpu.VMEM((2, _RR, W), jnp.float32),
            pltpu.VMEM((2 * _L,), jnp.float32),
            pltpu.SemaphoreType.DMA,
            pltpu.SemaphoreType.DMA,
            pltpu.SemaphoreType.DMA,
            pltpu.SemaphoreType.DMA,
        ],
    )(functools.partial(_sc_body, rows_half=rows_half, W=W))
    out = f(batchimg.reshape(B * C * H, W), batchmask.reshape(B * H, W),
            boost.reshape(B * C * H, W), ab)
    return out.reshape(B, C, H, W)
